# baseline (speedup 1.0000x reference)
import functools

import jax
import jax.numpy as jnp
from jax import lax
from jax.experimental import pallas as pl
from jax.experimental.pallas import tpu as pltpu
from jax.experimental.pallas import tpu_sc as plsc

N = 10000
K = 32
C_IN = 128
C_ADD = 16
C_MID = 16
C_TOT = C_IN + C_ADD

_NC, _NS = 2, 16
_NW = _NC * _NS
_ROWS = N * K
_ROWS_PER_W = _ROWS // _NW
_CHUNK = 80
_NCHUNK = _ROWS_PER_W // _CHUNK


def _sc_gather_body(table_hbm, idx_hbm, out_hbm,
                    idx_all, rows0, rows1, sem0, sem1):
    wid = lax.axis_index("s") * _NC + lax.axis_index("c")
    base = wid * _ROWS_PER_W
    rows = (rows0, rows1)
    sems = (sem0, sem1)
    pltpu.sync_copy(idx_hbm.at[pl.ds(base, _ROWS_PER_W)], idx_all)

    def _start(j, b):
        idx_sl = idx_all.at[pl.ds(j * _CHUNK, _CHUNK)]
        pltpu.async_copy(table_hbm.at[idx_sl], rows[b], sems[b])

    for b in range(2):
        _start(b, b)

    def body(i, carry):
        for b in range(2):
            j = 2 * i + b
            pltpu.make_async_copy(table_hbm.at[pl.ds(0, _CHUNK)],
                                  rows[b], sems[b]).wait()
            pltpu.sync_copy(rows[b], out_hbm.at[pl.ds(base + j * _CHUNK, _CHUNK)])

            @pl.when(j + 2 < _NCHUNK)
            def _():
                _start(j + 2, b)
        return carry

    lax.fori_loop(0, _NCHUNK // 2, body, 0)
    j_last = _NCHUNK - 1
    pltpu.make_async_copy(table_hbm.at[pl.ds(0, _CHUNK)],
                          rows[j_last % 2], sems[j_last % 2]).wait()
    pltpu.sync_copy(rows[j_last % 2],
                    out_hbm.at[pl.ds(base + j_last * _CHUNK, _CHUNK)])


def _sc_gather(table, idx_flat):
    mesh = plsc.VectorSubcoreMesh(core_axis_name="c", subcore_axis_name="s")
    fn = functools.partial(
        pl.kernel,
        out_type=jax.ShapeDtypeStruct((_ROWS, C_IN), jnp.float32),
        mesh=mesh,
        scratch_types=[
            pltpu.VMEM((_ROWS_PER_W,), jnp.int32),
            pltpu.VMEM((_CHUNK, C_IN), jnp.float32),
            pltpu.VMEM((_CHUNK, C_IN), jnp.float32),
            pltpu.SemaphoreType.DMA,
            pltpu.SemaphoreType.DMA,
        ],
    )(_sc_gather_body)
    return fn(table, idx_flat)


_PB = 80


def _tc_body(g_ref, w_ref, a_ref, o_ref):
    w = w_ref[...]
    r1 = lax.dot_general(w, g_ref[...], (((1,), (1,)), ((0,), (0,))),
                         preferred_element_type=jnp.float32)
    r2 = lax.dot_general(w, a_ref[...], (((1,), (1,)), ((0,), (0,))),
                         preferred_element_type=jnp.float32)
    o_ref[:, :, 0:C_IN] = r1
    o_ref[:, :, C_IN:C_TOT] = r2


_NCH = 5
_NH = N // _NCH


def _tc_compute(gathered, weightnet, additional, h):
    grid = _NH // _PB
    off = h * (_NH // _PB)
    return pl.pallas_call(
        _tc_body,
        grid=(grid,),
        in_specs=[
            pl.BlockSpec((_PB, K, C_IN), lambda i: (i + off, 0, 0)),
            pl.BlockSpec((_PB, K, C_MID), lambda i: (i + off, 0, 0)),
            pl.BlockSpec((_PB, K, C_ADD), lambda i: (i + off, 0, 0)),
        ],
        out_specs=pl.BlockSpec((_PB, C_MID, C_TOT), lambda i: (i, 0, 0)),
        out_shape=jax.ShapeDtypeStruct((_NH, C_MID, C_TOT), jnp.float32),
    )(gathered, weightnet, additional)


def kernel(input_features, neighbor_inds, weightnet, additional_features):
    table = input_features[0]
    idx_flat = neighbor_inds[0].astype(jnp.int32).reshape(_ROWS)
    gathered = _sc_gather(table, idx_flat).reshape(N, K, C_IN)
    parts = []
    for h in range(_NCH):
        out3 = _tc_compute(gathered, weightnet[0], additional_features[0], h)
        y = lax.optimization_barrier(jnp.transpose(out3, (2, 1, 0)))
        parts.append(y.reshape(C_TOT * C_MID, _NH).T)
    out = jnp.concatenate(parts, axis=0)
    return out.reshape(1, N, C_TOT * C_MID)


# speedup vs baseline: 1.0243x; 1.0243x over previous
import functools

import jax
import jax.numpy as jnp
from jax import lax
from jax.experimental import pallas as pl
from jax.experimental.pallas import tpu as pltpu
from jax.experimental.pallas import tpu_sc as plsc

N = 10000
K = 32
C_IN = 128
C_ADD = 16
C_MID = 16
C_TOT = C_IN + C_ADD

_NC, _NS = 2, 16
_NW = _NC * _NS
_ROWS = N * K
_ROWS_PER_W = _ROWS // _NW
_CHUNK = 80
_NCHUNK = _ROWS_PER_W // _CHUNK


def _sc_gather_body(table_hbm, idx_hbm, out_hbm,
                    idx_all, rows0, rows1, sem0, sem1):
    wid = lax.axis_index("s") * _NC + lax.axis_index("c")
    base = wid * _ROWS_PER_W
    rows = (rows0, rows1)
    sems = (sem0, sem1)
    pltpu.sync_copy(idx_hbm.at[pl.ds(base, _ROWS_PER_W)], idx_all)

    def _start(j, b):
        idx_sl = idx_all.at[pl.ds(j * _CHUNK, _CHUNK)]
        pltpu.async_copy(table_hbm.at[idx_sl], rows[b], sems[b])

    for b in range(2):
        _start(b, b)

    def body(i, carry):
        for b in range(2):
            j = 2 * i + b
            pltpu.make_async_copy(table_hbm.at[pl.ds(0, _CHUNK)],
                                  rows[b], sems[b]).wait()
            pltpu.sync_copy(rows[b], out_hbm.at[pl.ds(base + j * _CHUNK, _CHUNK)])

            @pl.when(j + 2 < _NCHUNK)
            def _():
                _start(j + 2, b)
        return carry

    lax.fori_loop(0, _NCHUNK // 2, body, 0)
    j_last = _NCHUNK - 1
    pltpu.make_async_copy(table_hbm.at[pl.ds(0, _CHUNK)],
                          rows[j_last % 2], sems[j_last % 2]).wait()
    pltpu.sync_copy(rows[j_last % 2],
                    out_hbm.at[pl.ds(base + j_last * _CHUNK, _CHUNK)])


def _sc_gather(table, idx_flat):
    mesh = plsc.VectorSubcoreMesh(core_axis_name="c", subcore_axis_name="s")
    fn = functools.partial(
        pl.kernel,
        out_type=jax.ShapeDtypeStruct((_ROWS, C_IN), jnp.float32),
        mesh=mesh,
        scratch_types=[
            pltpu.VMEM((_ROWS_PER_W,), jnp.int32),
            pltpu.VMEM((_CHUNK, C_IN), jnp.float32),
            pltpu.VMEM((_CHUNK, C_IN), jnp.float32),
            pltpu.SemaphoreType.DMA,
            pltpu.SemaphoreType.DMA,
        ],
    )(_sc_gather_body)
    return fn(table, idx_flat)


_PB = 80


def _tc_body(g_ref, w_ref, a_ref, o_ref):
    w = w_ref[...]
    r1 = lax.dot_general(w, g_ref[...], (((1,), (1,)), ((0,), (0,))),
                         preferred_element_type=jnp.float32)
    r2 = lax.dot_general(w, a_ref[...], (((1,), (1,)), ((0,), (0,))),
                         preferred_element_type=jnp.float32)
    o_ref[:, :, 0:C_IN] = r1
    o_ref[:, :, C_IN:C_TOT] = r2


_NCH = 25
_NH = N // _NCH


def _tc_compute(gathered, weightnet, additional, h):
    grid = _NH // _PB
    off = h * (_NH // _PB)
    return pl.pallas_call(
        _tc_body,
        grid=(grid,),
        in_specs=[
            pl.BlockSpec((_PB, K, C_IN), lambda i: (i + off, 0, 0)),
            pl.BlockSpec((_PB, K, C_MID), lambda i: (i + off, 0, 0)),
            pl.BlockSpec((_PB, K, C_ADD), lambda i: (i + off, 0, 0)),
        ],
        out_specs=pl.BlockSpec((_PB, C_MID, C_TOT), lambda i: (i, 0, 0)),
        out_shape=jax.ShapeDtypeStruct((_NH, C_MID, C_TOT), jnp.float32),
    )(gathered, weightnet, additional)


def kernel(input_features, neighbor_inds, weightnet, additional_features):
    table = input_features[0]
    idx_flat = neighbor_inds[0].astype(jnp.int32).reshape(_ROWS)
    gathered = _sc_gather(table, idx_flat).reshape(N, K, C_IN)
    parts = []
    for h in range(_NCH):
        out3 = _tc_compute(gathered, weightnet[0], additional_features[0], h)
        y = lax.optimization_barrier(jnp.transpose(out3, (2, 1, 0)))
        parts.append(y.reshape(C_TOT * C_MID, _NH).T)
    out = jnp.concatenate(parts, axis=0)
    return out.reshape(1, N, C_TOT * C_MID)
